# baseline (speedup 1.0000x reference)
import jax
import jax.numpy as jnp
from jax.experimental import pallas as pl
from jax.experimental.pallas import tpu as pltpu

_B = 64
_D = 32
_T = 1024
_K = 1024
_N = _B * _T
_COMMIT = 10.0


def _vq_body(x_ref, e_ref, out_ref, idx_ref, loss_ref, usage_ref,
             mask_acc, loss_acc):
    b = pl.program_id(0)

    X = x_ref[0]
    E = e_ref[...]

    e2 = jnp.sum(E * E, axis=1, keepdims=True)
    lane = jax.lax.broadcasted_iota(jnp.int32, (_K, _D + 8), 1)
    E_aug = (jax.lax.pad(E, 0.0, ((0, 0, 0), (0, 8, 0)))
             + jnp.where(lane == _D + 7,
                         jax.lax.broadcast_in_dim(-0.5 * e2, (_K, _D + 8),
                                                  (0, 1)),
                         0.0))
    X_aug = jnp.concatenate(
        [X, jnp.ones((8, _T), jnp.float32)], axis=0)
    S = jax.lax.dot_general(E_aug, X_aug, (((1,), (0,)), ((), ())),
                            preferred_element_type=jnp.float32)

    idx = jnp.argmax(S, axis=0)
    onehot = (jax.lax.broadcasted_iota(jnp.int32, (_K, _T), 0)
              == idx[None, :]).astype(jnp.float32)
    Q = jax.lax.dot_general(E, onehot, (((0,), (0,)), ((), ())),
                            preferred_element_type=jnp.float32)

    out_ref[0] = Q
    idx_ref[0, 0] = idx

    diff = Q - X
    sq = jnp.sum(diff * diff)
    cnt = jax.lax.dot_general(onehot, jnp.ones((_T, 8), jnp.float32),
                              (((1,), (0,)), ((), ())),
                              preferred_element_type=jnp.float32)

    @pl.when(b == 0)
    def _init():
        loss_acc[0, 0] = 0.0
        mask_acc[...] = jnp.zeros_like(mask_acc)

    loss_acc[0, 0] += sq
    mask_acc[...] += cnt

    @pl.when(b == pl.num_programs(0) - 1)
    def _fini():
        loss_ref[0, 0] = loss_acc[0, 0] * ((1.0 + _COMMIT) / float(_N * _D))
        usage_ref[0, 0] = jnp.sum(
            (mask_acc[:, 0:1] > 0.0).astype(jnp.float32)).astype(jnp.int32)


def kernel(inputs, embedding_weight):
    out, idx3, loss, usage = pl.pallas_call(
        _vq_body,
        grid=(_B,),
        in_specs=[
            pl.BlockSpec((1, _D, _T), lambda b: (b, 0, 0)),
            pl.BlockSpec((_K, _D), lambda b: (0, 0)),
        ],
        out_specs=[
            pl.BlockSpec((1, _D, _T), lambda b: (b, 0, 0)),
            pl.BlockSpec((1, 1, _T), lambda b: (b, 0, 0)),
            pl.BlockSpec(memory_space=pltpu.SMEM),
            pl.BlockSpec(memory_space=pltpu.SMEM),
        ],
        out_shape=[
            jax.ShapeDtypeStruct((_B, _D, _T), jnp.float32),
            jax.ShapeDtypeStruct((_B, 1, _T), jnp.int32),
            jax.ShapeDtypeStruct((1, 1), jnp.float32),
            jax.ShapeDtypeStruct((1, 1), jnp.int32),
        ],
        scratch_shapes=[
            pltpu.VMEM((_K, 8), jnp.float32),
            pltpu.SMEM((1, 1), jnp.float32),
        ],
    )(inputs, embedding_weight)
    return (out, loss[0, 0], usage[0, 0], idx3.reshape(_N, 1))


# speedup vs baseline: 1.2834x; 1.2834x over previous
import jax
import jax.numpy as jnp
from jax.experimental import pallas as pl
from jax.experimental.pallas import tpu as pltpu

_B = 64
_D = 32
_T = 1024
_K = 1024
_N = _B * _T
_COMMIT = 10.0
_BB = 2


def _vq_body(x_ref, e_ref, out_ref, idx_ref, loss_ref, usage_ref,
             mask_acc, loss_acc):
    b = pl.program_id(0)

    E = e_ref[...]
    e2 = jnp.sum(E * E, axis=1, keepdims=True)

    sq = 0.0
    used = None
    for i in range(_BB):
        X = x_ref[i]
        S = jax.lax.dot_general(E, X, (((1,), (0,)), ((), ())),
                                preferred_element_type=jnp.float32)
        S = S - 0.5 * e2

        idx = jnp.argmax(S, axis=0)
        onehot = (jax.lax.broadcasted_iota(jnp.int32, (_K, _T), 0)
                  == idx[None, :]).astype(jnp.float32)
        Q = jax.lax.dot_general(E, onehot, (((0,), (0,)), ((), ())),
                                preferred_element_type=jnp.float32)

        out_ref[i] = Q
        idx_ref[i, 0] = idx

        diff = Q - X
        sq = sq + jnp.sum(diff * diff)
        u = jnp.max(onehot, axis=1, keepdims=True)
        used = u if used is None else jnp.maximum(used, u)

    @pl.when(b == 0)
    def _init():
        loss_acc[0, 0] = 0.0
        mask_acc[...] = jnp.zeros_like(mask_acc)

    loss_acc[0, 0] += sq
    mask_acc[...] = jnp.maximum(mask_acc[...], used)

    @pl.when(b == pl.num_programs(0) - 1)
    def _fini():
        loss_ref[0, 0] = loss_acc[0, 0] * ((1.0 + _COMMIT) / float(_N * _D))
        usage_ref[0, 0] = jnp.sum(mask_acc[...]).astype(jnp.int32)


def kernel(inputs, embedding_weight):
    out, idx3, loss, usage = pl.pallas_call(
        _vq_body,
        grid=(_B // _BB,),
        in_specs=[
            pl.BlockSpec((_BB, _D, _T), lambda b: (b, 0, 0)),
            pl.BlockSpec((_K, _D), lambda b: (0, 0)),
        ],
        out_specs=[
            pl.BlockSpec((_BB, _D, _T), lambda b: (b, 0, 0)),
            pl.BlockSpec((_BB, 1, _T), lambda b: (b, 0, 0)),
            pl.BlockSpec(memory_space=pltpu.SMEM),
            pl.BlockSpec(memory_space=pltpu.SMEM),
        ],
        out_shape=[
            jax.ShapeDtypeStruct((_B, _D, _T), jnp.float32),
            jax.ShapeDtypeStruct((_B, 1, _T), jnp.int32),
            jax.ShapeDtypeStruct((1, 1), jnp.float32),
            jax.ShapeDtypeStruct((1, 1), jnp.int32),
        ],
        scratch_shapes=[
            pltpu.VMEM((_K, 1), jnp.float32),
            pltpu.SMEM((1, 1), jnp.float32),
        ],
    )(inputs, embedding_weight)
    return (out, loss[0, 0], usage[0, 0], idx3.reshape(_N, 1))


# speedup vs baseline: 1.3708x; 1.0680x over previous
import jax
import jax.numpy as jnp
from jax.experimental import pallas as pl
from jax.experimental.pallas import tpu as pltpu

_B = 64
_D = 32
_T = 1024
_K = 1024
_N = _B * _T
_COMMIT = 10.0
_BB = 4


def _vq_body(x_ref, e_ref, out_ref, idx_ref, loss_ref, usage_ref,
             mask_acc, loss_acc):
    b = pl.program_id(0)

    E = e_ref[...]
    e2 = jnp.sum(E * E, axis=1, keepdims=True)

    sq = 0.0
    used = None
    for i in range(_BB):
        X = x_ref[i]
        S = jax.lax.dot_general(E, X, (((1,), (0,)), ((), ())),
                                preferred_element_type=jnp.float32)
        S = S - 0.5 * e2

        idx = jnp.argmax(S, axis=0)
        onehot = (jax.lax.broadcasted_iota(jnp.int32, (_K, _T), 0)
                  == idx[None, :]).astype(jnp.float32)
        Q = jax.lax.dot_general(E, onehot, (((0,), (0,)), ((), ())),
                                preferred_element_type=jnp.float32)

        out_ref[i] = Q
        idx_ref[i, 0] = idx

        diff = Q - X
        sq = sq + jnp.sum(diff * diff)
        u = jnp.max(onehot, axis=1, keepdims=True)
        used = u if used is None else jnp.maximum(used, u)

    @pl.when(b == 0)
    def _init():
        loss_acc[0, 0] = 0.0
        mask_acc[...] = jnp.zeros_like(mask_acc)

    loss_acc[0, 0] += sq
    mask_acc[...] = jnp.maximum(mask_acc[...], used)

    @pl.when(b == pl.num_programs(0) - 1)
    def _fini():
        loss_ref[0, 0] = loss_acc[0, 0] * ((1.0 + _COMMIT) / float(_N * _D))
        usage_ref[0, 0] = jnp.sum(mask_acc[...]).astype(jnp.int32)


def kernel(inputs, embedding_weight):
    out, idx3, loss, usage = pl.pallas_call(
        _vq_body,
        grid=(_B // _BB,),
        in_specs=[
            pl.BlockSpec((_BB, _D, _T), lambda b: (b, 0, 0)),
            pl.BlockSpec((_K, _D), lambda b: (0, 0)),
        ],
        out_specs=[
            pl.BlockSpec((_BB, _D, _T), lambda b: (b, 0, 0)),
            pl.BlockSpec((_BB, 1, _T), lambda b: (b, 0, 0)),
            pl.BlockSpec(memory_space=pltpu.SMEM),
            pl.BlockSpec(memory_space=pltpu.SMEM),
        ],
        out_shape=[
            jax.ShapeDtypeStruct((_B, _D, _T), jnp.float32),
            jax.ShapeDtypeStruct((_B, 1, _T), jnp.int32),
            jax.ShapeDtypeStruct((1, 1), jnp.float32),
            jax.ShapeDtypeStruct((1, 1), jnp.int32),
        ],
        scratch_shapes=[
            pltpu.VMEM((_K, 1), jnp.float32),
            pltpu.SMEM((1, 1), jnp.float32),
        ],
    )(inputs, embedding_weight)
    return (out, loss[0, 0], usage[0, 0], idx3.reshape(_N, 1))


# speedup vs baseline: 1.4094x; 1.0282x over previous
import jax
import jax.numpy as jnp
from jax.experimental import pallas as pl
from jax.experimental.pallas import tpu as pltpu

_B = 64
_D = 32
_T = 1024
_K = 1024
_N = _B * _T
_COMMIT = 10.0
_BB = 8


def _vq_body(x_ref, e_ref, out_ref, idx_ref, loss_ref, usage_ref,
             mask_acc, loss_acc):
    b = pl.program_id(0)

    E = e_ref[...]
    e2 = jnp.sum(E * E, axis=1, keepdims=True)

    sq = 0.0
    used = None
    for i in range(_BB):
        X = x_ref[i]
        S = jax.lax.dot_general(E, X, (((1,), (0,)), ((), ())),
                                preferred_element_type=jnp.float32)
        S = S - 0.5 * e2

        idx = jnp.argmax(S, axis=0)
        onehot = (jax.lax.broadcasted_iota(jnp.int32, (_K, _T), 0)
                  == idx[None, :]).astype(jnp.float32)
        Q = jax.lax.dot_general(E, onehot, (((0,), (0,)), ((), ())),
                                preferred_element_type=jnp.float32)

        out_ref[i] = Q
        idx_ref[i, 0] = idx

        diff = Q - X
        sq = sq + jnp.sum(diff * diff)
        u = jnp.max(onehot, axis=1, keepdims=True)
        used = u if used is None else jnp.maximum(used, u)

    @pl.when(b == 0)
    def _init():
        loss_acc[0, 0] = 0.0
        mask_acc[...] = jnp.zeros_like(mask_acc)

    loss_acc[0, 0] += sq
    mask_acc[...] = jnp.maximum(mask_acc[...], used)

    @pl.when(b == pl.num_programs(0) - 1)
    def _fini():
        loss_ref[0, 0] = loss_acc[0, 0] * ((1.0 + _COMMIT) / float(_N * _D))
        usage_ref[0, 0] = jnp.sum(mask_acc[...]).astype(jnp.int32)


def kernel(inputs, embedding_weight):
    out, idx3, loss, usage = pl.pallas_call(
        _vq_body,
        grid=(_B // _BB,),
        in_specs=[
            pl.BlockSpec((_BB, _D, _T), lambda b: (b, 0, 0)),
            pl.BlockSpec((_K, _D), lambda b: (0, 0)),
        ],
        out_specs=[
            pl.BlockSpec((_BB, _D, _T), lambda b: (b, 0, 0)),
            pl.BlockSpec((_BB, 1, _T), lambda b: (b, 0, 0)),
            pl.BlockSpec(memory_space=pltpu.SMEM),
            pl.BlockSpec(memory_space=pltpu.SMEM),
        ],
        out_shape=[
            jax.ShapeDtypeStruct((_B, _D, _T), jnp.float32),
            jax.ShapeDtypeStruct((_B, 1, _T), jnp.int32),
            jax.ShapeDtypeStruct((1, 1), jnp.float32),
            jax.ShapeDtypeStruct((1, 1), jnp.int32),
        ],
        scratch_shapes=[
            pltpu.VMEM((_K, 1), jnp.float32),
            pltpu.SMEM((1, 1), jnp.float32),
        ],
    )(inputs, embedding_weight)
    return (out, loss[0, 0], usage[0, 0], idx3.reshape(_N, 1))
